# baseline (speedup 1.0000x reference)
import functools

import jax
import jax.numpy as jnp
from jax import lax
from jax.experimental import pallas as pl
from jax.experimental.pallas import tpu as pltpu
from jax.experimental.pallas import tpu_sc as plsc

EMBEDDING_DIM = 64

_NUM_CORES = 2
_NUM_SUBCORES = 16
_NUM_WORKERS = _NUM_CORES * _NUM_SUBCORES

_CHUNK = 320


@functools.partial(jax.jit, static_argnames=("num_indices",))
def _embedding_gather(weight, flat_ids, *, num_indices):
    b_per_w = num_indices // _NUM_WORKERS
    n_chunks = b_per_w // _CHUNK
    assert n_chunks % 2 == 0
    mesh = plsc.VectorSubcoreMesh(core_axis_name="c", subcore_axis_name="s")

    @functools.partial(
        pl.kernel,
        mesh=mesh,
        compiler_params=pltpu.CompilerParams(use_tc_tiling_on_sc=False),
        out_type=jax.ShapeDtypeStruct((num_indices, EMBEDDING_DIM), jnp.float32),
        scratch_types=[
            pltpu.VMEM((b_per_w,), jnp.int32),
            *[pltpu.VMEM((_CHUNK, EMBEDDING_DIM), jnp.float32) for _ in range(2)],
            *[pltpu.SemaphoreType.DMA for _ in range(4)],
        ],
    )
    def gather_kernel(table_hbm, idx_hbm, out_hbm, idx_v, *scr):
        rows = scr[0:2]
        gsem = scr[2:4]
        osem = scr[4:6]

        wid = lax.axis_index("s") * _NUM_CORES + lax.axis_index("c")
        base = wid * b_per_w

        pltpu.sync_copy(idx_hbm.at[pl.ds(base, b_per_w)], idx_v)

        def issue_gathers(c, b):
            def group(j, carry):
                idv = idx_v[pl.ds(c * _CHUNK + j * 16, 16)]
                for l in range(16):
                    pltpu.async_copy(
                        table_hbm.at[idv[l]],
                        rows[b].at[j * 16 + l],
                        gsem[b],
                    )
                return carry

            lax.fori_loop(0, _CHUNK // 16, group, 0)

        def drain_gathers(b):
            def one(i, carry):
                pltpu.make_async_copy(
                    table_hbm.at[0], rows[b].at[0], gsem[b]
                ).wait()
                return carry

            lax.fori_loop(0, _CHUNK, one, 0)

        def start_out(c, b):
            return pltpu.async_copy(
                rows[b], out_hbm.at[pl.ds(base + c * _CHUNK, _CHUNK)], osem[b]
            )

        def wait_out(b):
            pltpu.make_async_copy(
                out_hbm.at[pl.ds(0, _CHUNK)], rows[b], osem[b]
            ).wait()

        def body(t, carry):
            c0 = 2 * t
            c1 = c0 + 1

            @pl.when(t > 0)
            def _():
                wait_out(0)

            issue_gathers(c0, 0)

            @pl.when(t > 0)
            def _():
                wait_out(1)

            issue_gathers(c1, 1)
            drain_gathers(0)
            start_out(c0, 0)
            drain_gathers(1)
            start_out(c1, 1)
            return carry

        lax.fori_loop(0, n_chunks // 2, body, 0)
        wait_out(0)
        wait_out(1)

    return gather_kernel(weight, flat_ids)


def kernel(token_ids, weight):
    batch, seq = token_ids.shape
    num_rows, dim = weight.shape
    flat = token_ids.reshape(-1).astype(jnp.int32)
    out = _embedding_gather(weight, flat, num_indices=batch * seq)
    return out.reshape(batch, seq, dim)


# speedup vs baseline: 1.5503x; 1.5503x over previous
import functools

import jax
import jax.numpy as jnp
from jax import lax
from jax.experimental import pallas as pl
from jax.experimental.pallas import tpu as pltpu
from jax.experimental.pallas import tpu_sc as plsc

EMBEDDING_DIM = 64

_NUM_CORES = 2
_NUM_SUBCORES = 16
_NUM_WORKERS = _NUM_CORES * _NUM_SUBCORES

_CHUNK = 320


@functools.partial(jax.jit, static_argnames=("num_indices",))
def _embedding_gather(weight, flat_ids, *, num_indices):
    b_per_w = num_indices // _NUM_WORKERS
    n_chunks = b_per_w // _CHUNK
    assert n_chunks % 2 == 0
    mesh = plsc.VectorSubcoreMesh(core_axis_name="c", subcore_axis_name="s")

    @functools.partial(
        pl.kernel,
        mesh=mesh,
        out_type=jax.ShapeDtypeStruct((num_indices, EMBEDDING_DIM), jnp.float32),
        scratch_types=[
            pltpu.VMEM((b_per_w,), jnp.int32),
            *[pltpu.VMEM((_CHUNK, EMBEDDING_DIM), jnp.float32) for _ in range(2)],
            *[pltpu.SemaphoreType.DMA for _ in range(4)],
        ],
    )
    def gather_kernel(table_hbm, idx_hbm, out_hbm, idx_v, *scr):
        rows = scr[0:2]
        gsem = scr[2:4]
        osem = scr[4:6]

        wid = lax.axis_index("s") * _NUM_CORES + lax.axis_index("c")
        base = wid * b_per_w

        pltpu.sync_copy(idx_hbm.at[pl.ds(base, b_per_w)], idx_v)

        def issue_gathers(c, b):
            def group(j, carry):
                idv = idx_v[pl.ds(c * _CHUNK + j * 16, 16)]
                for l in range(16):
                    pltpu.async_copy(
                        table_hbm.at[idv[l]],
                        rows[b].at[j * 16 + l],
                        gsem[b],
                    )
                return carry

            lax.fori_loop(0, _CHUNK // 16, group, 0)

        def drain_gathers(b):
            def one(i, carry):
                pltpu.make_async_copy(
                    table_hbm.at[0], rows[b].at[0], gsem[b]
                ).wait()
                return carry

            lax.fori_loop(0, _CHUNK, one, 0)

        def start_out(c, b):
            return pltpu.async_copy(
                rows[b], out_hbm.at[pl.ds(base + c * _CHUNK, _CHUNK)], osem[b]
            )

        def wait_out(b):
            pltpu.make_async_copy(
                out_hbm.at[pl.ds(0, _CHUNK)], rows[b], osem[b]
            ).wait()

        def body(t, carry):
            c0 = 2 * t
            c1 = c0 + 1

            @pl.when(t > 0)
            def _():
                wait_out(0)

            issue_gathers(c0, 0)

            @pl.when(t > 0)
            def _():
                wait_out(1)

            issue_gathers(c1, 1)
            drain_gathers(0)
            start_out(c0, 0)
            drain_gathers(1)
            start_out(c1, 1)
            return carry

        lax.fori_loop(0, n_chunks // 2, body, 0)
        wait_out(0)
        wait_out(1)

    return gather_kernel(weight, flat_ids)


def kernel(token_ids, weight):
    batch, seq = token_ids.shape
    num_rows, dim = weight.shape
    flat = token_ids.reshape(-1).astype(jnp.int32)
    out = _embedding_gather(weight, flat, num_indices=batch * seq)
    return out.reshape(batch, seq, dim)
